# baseline (speedup 1.0000x reference)
import functools

import jax
import jax.numpy as jnp
from jax import lax
from jax.experimental import pallas as pl
from jax.experimental.pallas import tpu as pltpu
from jax.experimental.pallas import tpu_sc as plsc

CAP = 100000
D = 128
BN = 1024
KN = 512
MOM = 0.5

_COPY_ROWS = 4000


_NBUF = 4


def _make_sc_gather2(n_idx, ch, same_idx, group=20):
  _GROUP = group
  info = plsc.get_sparse_core_info()
  nw = info.num_cores * info.num_subcores
  n_per = n_idx // nw
  assert n_per * nw == n_idx
  assert n_per % ch == 0 and ch % 8 == 0 and ch <= 128
  nchunks = n_per // ch
  pipelined = nchunks % _GROUP == 0 and nchunks >= _GROUP

  mesh = plsc.VectorSubcoreMesh(core_axis_name="c", subcore_axis_name="s")
  scratch = [pltpu.VMEM((n_per,), jnp.int32)]
  if not same_idx:
    scratch.append(pltpu.VMEM((n_per,), jnp.int32))
  nbuf = _NBUF if pipelined else 1
  scratch += [pltpu.VMEM((ch, D), jnp.float32) for _ in range(2 * nbuf)]
  scratch += [pltpu.SemaphoreType.DMA for _ in range(4 * nbuf)]

  @functools.partial(
      pl.kernel,
      mesh=mesh,
      out_type=(
          jax.ShapeDtypeStruct((n_idx, D), jnp.float32),
          jax.ShapeDtypeStruct((n_idx, D), jnp.float32),
      ),
      scratch_types=tuple(scratch),
  )
  def gk(tab_a, tab_b, idxr_a, idxr_b, out_a, out_b, *scr):
    wid = lax.axis_index("s") * info.num_cores + lax.axis_index("c")
    base0 = wid * n_per

    pos = 0
    idx_all_a = scr[pos]
    pos += 1
    if same_idx:
      idx_all_b = idx_all_a
    else:
      idx_all_b = scr[pos]
      pos += 1
    bufs_a = scr[pos:pos + nbuf]
    bufs_b = scr[pos + nbuf:pos + 2 * nbuf]
    pos += 2 * nbuf
    gsem_a = scr[pos:pos + nbuf]
    gsem_b = scr[pos + nbuf:pos + 2 * nbuf]
    ssem_a = scr[pos + 2 * nbuf:pos + 3 * nbuf]
    ssem_b = scr[pos + 3 * nbuf:pos + 4 * nbuf]

    pltpu.sync_copy(idxr_a.at[pl.ds(base0, n_per)], idx_all_a)
    if not same_idx:
      pltpu.sync_copy(idxr_b.at[pl.ds(base0, n_per)], idx_all_b)

    def start_gather(c, b):
      iva = idx_all_a.at[pl.ds(pl.multiple_of(c * ch, 8), ch)]
      ivb = idx_all_b.at[pl.ds(pl.multiple_of(c * ch, 8), ch)]
      return (pltpu.async_copy(tab_a.at[iva], bufs_a[b], gsem_a[b]),
              pltpu.async_copy(tab_b.at[ivb], bufs_b[b], gsem_b[b]))

    def start_store(c, b):
      dst = pl.multiple_of(base0 + c * ch, 8)
      return (pltpu.async_copy(bufs_a[b], out_a.at[pl.ds(dst, ch)], ssem_a[b]),
              pltpu.async_copy(bufs_b[b], out_b.at[pl.ds(dst, ch)], ssem_b[b]))

    if not pipelined:
      def body(c, carry):
        ga, gb = start_gather(c, 0)
        ga.wait()
        gb.wait()
        sa, sb = start_store(c, 0)
        sa.wait()
        sb.wait()
        return carry

      lax.fori_loop(0, nchunks, body, 0)
      return

    def group(g, carry):
      c0 = g * _GROUP
      gobjs = {}
      sobjs = {}
      gobjs[0] = start_gather(c0 + 0, 0)
      gobjs[1] = start_gather(c0 + 1, 1)
      for j in range(_GROUP):
        ga, gb = gobjs.pop(j)
        ga.wait()
        gb.wait()
        if j + 2 < _GROUP:
          if j - 2 >= 0:
            sa, sb = sobjs.pop(j - 2)
            sa.wait()
            sb.wait()
          gobjs[j + 2] = start_gather(c0 + j + 2, (j + 2) % _NBUF)
        sobjs[j] = start_store(c0 + j, j % _NBUF)
      for j in sorted(sobjs):
        sa, sb = sobjs[j]
        sa.wait()
        sb.wait()
      return carry

    lax.fori_loop(0, nchunks // _GROUP, group, 0)

  return gk


def _make_sc_gather1(n_idx, ch, group, nbuf=6, ga=4, patch=False):
  info = plsc.get_sparse_core_info()
  nw = info.num_cores * info.num_subcores
  n_per = n_idx // nw
  assert n_per * nw == n_idx and n_per % ch == 0
  nchunks = n_per // ch
  assert nchunks % group == 0 and ga < nbuf <= group
  ppt = BN // info.num_subcores

  mesh = plsc.VectorSubcoreMesh(core_axis_name="c", subcore_axis_name="s")
  scratch = [pltpu.VMEM((n_per,), jnp.int32)]
  scratch += [pltpu.VMEM((ch, D), jnp.float32) for _ in range(nbuf)]
  scratch += [pltpu.SemaphoreType.DMA for _ in range(2 * nbuf)]
  if patch:
    scratch += [pltpu.VMEM((ppt,), jnp.int32),
                pltpu.VMEM((ppt, D), jnp.float32),
                pltpu.SemaphoreType.DMA]

  @functools.partial(
      pl.kernel,
      mesh=mesh,
      out_type=jax.ShapeDtypeStruct((n_idx, D), jnp.float32),
      scratch_types=tuple(scratch),
  )
  def gk(tab, *rest):
    if patch:
      updf, idxp, idxr, out = rest[:4]
      scr = rest[4:]
    else:
      idxr, out = rest[:2]
      scr = rest[2:]
    wid = lax.axis_index("s") * info.num_cores + lax.axis_index("c")
    base0 = wid * n_per
    idx_all = scr[0]
    bufs = scr[1:1 + nbuf]
    gsem = scr[1 + nbuf:1 + 2 * nbuf]
    ssem = scr[1 + 2 * nbuf:1 + 3 * nbuf]

    if patch:
      pidx, prow, psem = scr[1 + 3 * nbuf:1 + 3 * nbuf + 3]
      pbase = lax.axis_index("s") * ppt
      pltpu.sync_copy(idxp.at[pl.ds(pbase, ppt)], pidx)
      pltpu.sync_copy(updf.at[pl.ds(pbase, ppt)], prow)
      pltpu.async_copy(prow, tab.at[pidx], psem).wait()
      plsc.subcore_barrier()

    pltpu.sync_copy(idxr.at[pl.ds(base0, n_per)], idx_all)

    def start_gather(c, b):
      iv = idx_all.at[pl.ds(pl.multiple_of(c * ch, 8), ch)]
      return pltpu.async_copy(tab.at[iv], bufs[b], gsem[b])

    def start_store(c, b):
      dst = pl.multiple_of(base0 + c * ch, 8)
      return pltpu.async_copy(bufs[b], out.at[pl.ds(dst, ch)], ssem[b])

    def grp(g, carry):
      c0 = g * group
      gobjs = {k: start_gather(c0 + k, k % nbuf) for k in range(ga)}
      sobjs = {}
      for j in range(group):
        gobjs.pop(j).wait()
        if j + ga < group:
          if j + ga - nbuf >= 0:
            sobjs.pop(j + ga - nbuf).wait()
          gobjs[j + ga] = start_gather(c0 + j + ga, (j + ga) % nbuf)
        sobjs[j] = start_store(c0 + j, j % nbuf)
      for j in sorted(sobjs):
        sobjs[j].wait()
      return carry

    lax.fori_loop(0, nchunks // group, grp, 0)

  return gk


def _chain_body(ic_ref, ir_ref, old_s_ref, old_t_ref, f_s_ref, f_t_ref,
                updf_s_ref, updf_t_ref):
  ic = ic_ref[...]
  ir = ir_ref[...]
  eq = ic == ir
  jj = lax.broadcasted_iota(jnp.int32, (BN, BN), 1)
  ii = lax.broadcasted_iota(jnp.int32, (BN, BN), 0)
  eqlt = eq & (jj < ii)
  pred = jnp.max(jnp.where(eqlt, jj, -1), axis=1, keepdims=True)
  depth = jnp.sum(eqlt.astype(jnp.int32), axis=1, keepdims=True)
  fin = jnp.max(jnp.where(eq & (jj >= ii), jj, -1), axis=1, keepdims=True)
  psel = (jj == pred).astype(jnp.float32)
  fsel = (jj == fin).astype(jnp.float32)
  maxd = jnp.max(depth)

  old_s = old_s_ref[...]
  old_t = old_t_ref[...]
  f_s = f_s_ref[...]
  f_t = f_t_ref[...]

  def one_bank(d, upd, old, f):
    prev = jnp.where(depth == 0, old,
                     jnp.dot(psel, upd, preferred_element_type=jnp.float32))
    v = MOM * prev + (1.0 - MOM) * f
    r = v * lax.rsqrt(jnp.sum(v * v, axis=1, keepdims=True))
    return jnp.where(depth == d, r, upd)

  def cond(carry):
    return carry[0] <= maxd

  def body(carry):
    d, us, ut = carry
    return d + 1, one_bank(d, us, old_s, f_s), one_bank(d, ut, old_t, f_t)

  zeros = jnp.zeros((BN, D), jnp.float32)
  _, upd_s, upd_t = lax.while_loop(cond, body, (0, zeros, zeros))

  updf_s_ref[...] = jnp.dot(fsel, upd_s, preferred_element_type=jnp.float32)
  updf_t_ref[...] = jnp.dot(fsel, upd_t, preferred_element_type=jnp.float32)


def _chain(idx, old_s, old_t, f_s, f_t):
  return pl.pallas_call(
      _chain_body,
      out_shape=(
          jax.ShapeDtypeStruct((BN, D), jnp.float32),
          jax.ShapeDtypeStruct((BN, D), jnp.float32),
      ),
  )(idx.reshape(BN, 1), idx.reshape(1, BN), old_s, old_t, f_s, f_t)


def _copy_body(lay_ref, in_blk, out_blk):
  del lay_ref
  out_blk[...] = in_blk[0]


def _copy_layer(memory, layer):
  lay = jnp.asarray(layer, jnp.int32).reshape(1)
  grid_spec = pltpu.PrefetchScalarGridSpec(
      num_scalar_prefetch=1,
      grid=(CAP // _COPY_ROWS,),
      in_specs=[
          pl.BlockSpec((1, _COPY_ROWS, D), lambda i, lay: (lay[0], i, 0)),
      ],
      out_specs=pl.BlockSpec((_COPY_ROWS, D), lambda i, lay: (i, 0)),
  )
  return pl.pallas_call(
      _copy_body,
      grid_spec=grid_spec,
      out_shape=jax.ShapeDtypeStruct((CAP, D), jnp.float32),
  )(lay, memory)


def kernel(f_s, f_t, s_layer, t_layer, idx, contrast_idx, memory_v1,
           memory_v2):
  idx = idx.astype(jnp.int32)
  soff = jnp.asarray(s_layer, jnp.int32) * CAP
  toff = jnp.asarray(t_layer, jnp.int32) * CAP
  mem1f = memory_v1.reshape(4 * CAP, D)
  mem2f = memory_v2.reshape(4 * CAP, D)
  fi2 = jnp.concatenate([idx[:, None], contrast_idx.astype(jnp.int32)],
                        axis=1)
  fidx = jnp.swapaxes(fi2, 0, 1).reshape(-1)

  g_small = _make_sc_gather2(BN, 32, False)
  old_s, old_t = g_small(mem1f, mem2f, idx + soff, idx + toff)

  updf_s, updf_t = _chain(idx, old_s, old_t, f_s, f_t)

  g_big = _make_sc_gather1((KN + 1) * BN, 96, 19, patch=True)
  cref_s = jax.new_ref(_copy_layer(memory_v1, s_layer))
  w_s = g_big(cref_s, updf_s, idx, fidx)
  cref_t = jax.new_ref(_copy_layer(memory_v2, t_layer))
  w_t = g_big(cref_t, updf_t, idx, fidx)
  return (jnp.swapaxes(w_s.reshape(KN + 1, BN, D), 0, 1),
          jnp.swapaxes(w_t.reshape(KN + 1, BN, D), 0, 1))
